# baseline (speedup 1.0000x reference)
import jax
import jax.numpy as jnp
from jax.experimental import pallas as pl

_VARIANCE = 1e-11


def _ln_body(x_ref, pos_ref, gamma_ref, beta_ref, out_ref):
    xb = x_ref[...]
    pe = pos_ref[...]
    v = xb + pe[:, None, :]
    u = jnp.mean(v, axis=-1, keepdims=True)
    d = v - u
    s = jnp.mean(d * d, axis=-1, keepdims=True)
    inv = jax.lax.rsqrt(s + _VARIANCE)
    out_ref[...] = d * inv * gamma_ref[0][None, None, :] + beta_ref[0][None, None, :]


def kernel(x, pos_table, gamma, beta):
    S, B, D = x.shape
    BS = 256
    grid = (S // BS,)
    gamma2 = gamma.reshape(1, D)
    beta2 = beta.reshape(1, D)
    return pl.pallas_call(
        _ln_body,
        grid=grid,
        in_specs=[
            pl.BlockSpec((BS, B, D), lambda i: (i, 0, 0)),
            pl.BlockSpec((BS, D), lambda i: (i, 0)),
            pl.BlockSpec((1, D), lambda i: (0, 0)),
            pl.BlockSpec((1, D), lambda i: (0, 0)),
        ],
        out_specs=pl.BlockSpec((BS, B, D), lambda i: (i, 0, 0)),
        out_shape=jax.ShapeDtypeStruct((S, B, D), x.dtype),
    )(x, pos_table, gamma2, beta2)


# speedup vs baseline: 1.1119x; 1.1119x over previous
import jax
import jax.numpy as jnp
from jax.experimental import pallas as pl

_VARIANCE = 1e-11


def _ln_body(x_ref, pos_ref, gamma_ref, beta_ref, out_ref):
    BS, B, D = x_ref.shape
    R = BS * B
    xb = x_ref.reshape(R, D)[...]
    pe = jnp.repeat(pos_ref[...], B, axis=0)
    v = xb + pe
    u = jnp.mean(v, axis=-1, keepdims=True)
    q = jnp.mean(v * v, axis=-1, keepdims=True)
    inv = jax.lax.rsqrt(q - u * u + _VARIANCE)
    g = gamma_ref[0][None, :]
    bt = beta_ref[0][None, :]
    out_ref.reshape(R, D)[...] = (v * inv - u * inv) * g + bt


def kernel(x, pos_table, gamma, beta):
    S, B, D = x.shape
    BS = 256
    grid = (S // BS,)
    gamma2 = gamma.reshape(1, D)
    beta2 = beta.reshape(1, D)
    return pl.pallas_call(
        _ln_body,
        grid=grid,
        in_specs=[
            pl.BlockSpec((BS, B, D), lambda i: (i, 0, 0)),
            pl.BlockSpec((BS, D), lambda i: (i, 0)),
            pl.BlockSpec((1, D), lambda i: (0, 0)),
            pl.BlockSpec((1, D), lambda i: (0, 0)),
        ],
        out_specs=pl.BlockSpec((BS, B, D), lambda i: (i, 0, 0)),
        out_shape=jax.ShapeDtypeStruct((S, B, D), x.dtype),
    )(x, pos_table, gamma2, beta2)
